# baseline (speedup 1.0000x reference)
import functools

import jax
import jax.numpy as jnp
from jax import lax
from jax.experimental import pallas as pl
from jax.experimental.pallas import tpu as pltpu
from jax.experimental.pallas import tpu_sc as plsc

NUM_CORES = 2
NUM_SUBCORES = 16
NUM_WORKERS = NUM_CORES * NUM_SUBCORES
LANES = 16

BATCH = 1024
SEQ = 200
D = 64
HALF = SEQ // 2
B_PER_W = BATCH // NUM_WORKERS


def _embed_kernel(ids_hbm, table_hbm, pos_hbm, out_hbm,
                  pos_v, idx_v, tok_v, sem_g, sem_o):
    wid = lax.axis_index("s") * NUM_CORES + lax.axis_index("c")
    base = wid * B_PER_W

    pltpu.sync_copy(pos_hbm.at[pl.ds(0, SEQ)], pos_v)

    def body(b_local, carry):
        b = base + b_local
        pltpu.sync_copy(ids_hbm.at[b], idx_v)
        g0 = pltpu.async_copy(table_hbm.at[idx_v.at[0]],
                              tok_v.at[pl.ds(0, HALF)], sem_g)
        g1 = pltpu.async_copy(table_hbm.at[idx_v.at[1]],
                              tok_v.at[pl.ds(HALF, HALF)], sem_g)
        g0.wait()
        g1.wait()

        def add_row(i, c):
            for j in range(D // LANES):
                sl = pl.ds(j * LANES, LANES)
                plsc.addupdate(tok_v.at[i, sl], pos_v[i, sl])
            return c

        lax.fori_loop(0, SEQ, add_row, 0, unroll=2)

        pltpu.async_copy(tok_v, out_hbm.at[b], sem_o).wait()
        return carry

    lax.fori_loop(0, B_PER_W, body, 0)


@jax.jit
def kernel(token_ids, token_table, pos_table):
    ids = token_ids.astype(jnp.int32).reshape(BATCH, 2, HALF)
    mesh = plsc.VectorSubcoreMesh(core_axis_name="c", subcore_axis_name="s")
    run = functools.partial(
        pl.kernel,
        out_type=jax.ShapeDtypeStruct((BATCH, SEQ, D), jnp.float32),
        mesh=mesh,
        scratch_types=[
            pltpu.VMEM((SEQ, D), jnp.float32),
            pltpu.VMEM((2, HALF), jnp.int32),
            pltpu.VMEM((SEQ, D), jnp.float32),
            pltpu.SemaphoreType.DMA,
            pltpu.SemaphoreType.DMA,
        ],
        compiler_params=pltpu.CompilerParams(use_tc_tiling_on_sc=False),
    )(_embed_kernel)
    return run(ids, token_table, pos_table)


# speedup vs baseline: 1.0723x; 1.0723x over previous
import functools

import jax
import jax.numpy as jnp
from jax import lax
from jax.experimental import pallas as pl
from jax.experimental.pallas import tpu as pltpu
from jax.experimental.pallas import tpu_sc as plsc

NUM_CORES = 2
NUM_SUBCORES = 16
NUM_WORKERS = NUM_CORES * NUM_SUBCORES
LANES = 16

BATCH = 1024
SEQ = 200
D = 64
HALF = SEQ // 2
B_PER_W = BATCH // NUM_WORKERS
CHUNK = 4
NBUF = 2
NSTEPS = B_PER_W // CHUNK


def _embed_kernel(ids_hbm, table_hbm, pos_hbm, out_hbm,
                  pos_v, idx_v, tok_v, sem_g0, sem_g1, sem_o0, sem_o1):
    sems_g = (sem_g0, sem_g1)
    sems_o = (sem_o0, sem_o1)
    wid = lax.axis_index("s") * NUM_CORES + lax.axis_index("c")
    base = wid * B_PER_W

    pltpu.sync_copy(pos_hbm.at[pl.ds(0, SEQ)], pos_v)

    def stage(g):
        slot = g % NBUF
        b0 = base + g * CHUNK
        pltpu.sync_copy(ids_hbm.at[pl.ds(b0, CHUNK)], idx_v.at[slot])
        handles = []
        for c in range(CHUNK):
            for h in range(2):
                handles.append(pltpu.async_copy(
                    table_hbm.at[idx_v.at[slot, c, h]],
                    tok_v.at[slot, c, pl.ds(h * HALF, HALF)],
                    sems_g[slot]))
        return handles

    gather_h = {0: stage(0)}
    out_h = {}
    for g in range(NSTEPS):
        slot = g % NBUF
        if g + 1 < NSTEPS:
            if g + 1 - NBUF >= 0:
                out_h.pop(g + 1 - NBUF).wait()
            gather_h[g + 1] = stage(g + 1)
        for h in gather_h.pop(g):
            h.wait()

        tok = tok_v.at[slot]

        def add_row(i, carry):
            for j in range(D // LANES):
                sl = pl.ds(j * LANES, LANES)
                p = pos_v[i, sl]
                for c in range(CHUNK):
                    plsc.addupdate(tok.at[c, i, sl], p)
            return carry

        lax.fori_loop(0, SEQ, add_row, 0)

        b0 = base + g * CHUNK
        out_h[g] = pltpu.async_copy(tok, out_hbm.at[pl.ds(b0, CHUNK)],
                                    sems_o[slot])
    for h in out_h.values():
        h.wait()


@jax.jit
def kernel(token_ids, token_table, pos_table):
    ids = token_ids.astype(jnp.int32).reshape(BATCH, 2, HALF)
    mesh = plsc.VectorSubcoreMesh(core_axis_name="c", subcore_axis_name="s")
    run = functools.partial(
        pl.kernel,
        out_type=jax.ShapeDtypeStruct((BATCH, SEQ, D), jnp.float32),
        mesh=mesh,
        scratch_types=[
            pltpu.VMEM((SEQ, D), jnp.float32),
            pltpu.VMEM((NBUF, CHUNK, 2, HALF), jnp.int32),
            pltpu.VMEM((NBUF, CHUNK, SEQ, D), jnp.float32),
            pltpu.SemaphoreType.DMA,
            pltpu.SemaphoreType.DMA,
            pltpu.SemaphoreType.DMA,
            pltpu.SemaphoreType.DMA,
        ],
        compiler_params=pltpu.CompilerParams(use_tc_tiling_on_sc=False),
    )(_embed_kernel)
    return run(ids, token_table, pos_table)


# speedup vs baseline: 1.0755x; 1.0030x over previous
import functools

import jax
import jax.numpy as jnp
from jax import lax
from jax.experimental import pallas as pl
from jax.experimental.pallas import tpu as pltpu
from jax.experimental.pallas import tpu_sc as plsc

NUM_CORES = 2
NUM_SUBCORES = 16
NUM_WORKERS = NUM_CORES * NUM_SUBCORES
LANES = 16

BATCH = 1024
SEQ = 200
D = 64
SPLITS = ((0, 104), (104, 96))
B_PER_W = BATCH // NUM_WORKERS
CHUNK = 4
NBUF = 2
NSTEPS = B_PER_W // CHUNK


def _embed_kernel(ids_hbm, table_hbm, pos_hbm, out_hbm,
                  pos_v, idx_v, tok_v, sem_g0, sem_g1, sem_o0, sem_o1):
    sems_g = (sem_g0, sem_g1)
    sems_o = (sem_o0, sem_o1)
    wid = lax.axis_index("s") * NUM_CORES + lax.axis_index("c")
    base = wid * B_PER_W

    pltpu.sync_copy(pos_hbm.at[pl.ds(0, SEQ)], pos_v)

    def stage(g):
        slot = g % NBUF
        b0 = base + g * CHUNK
        pltpu.sync_copy(ids_hbm.at[pl.ds(b0, CHUNK)], idx_v.at[slot])
        handles = []
        for c in range(CHUNK):
            for off, n in SPLITS:
                handles.append(pltpu.async_copy(
                    table_hbm.at[idx_v.at[slot, c, pl.ds(off, n)]],
                    tok_v.at[slot, c, pl.ds(off, n)],
                    sems_g[slot]))
        return handles

    gather_h = {0: stage(0)}
    out_h = {}
    for g in range(NSTEPS):
        slot = g % NBUF
        if g + 1 < NSTEPS:
            if g + 1 - NBUF >= 0:
                out_h.pop(g + 1 - NBUF).wait()
            gather_h[g + 1] = stage(g + 1)
        for h in gather_h.pop(g):
            h.wait()

        tok = tok_v.at[slot]

        def add_row(i, carry):
            for j in range(D // LANES):
                sl = pl.ds(j * LANES, LANES)
                p = pos_v[i, sl]
                for c in range(CHUNK):
                    plsc.addupdate(tok.at[c, i, sl], p)
            return carry

        lax.fori_loop(0, SEQ, add_row, 0)

        b0 = base + g * CHUNK
        out_h[g] = pltpu.async_copy(tok, out_hbm.at[pl.ds(b0, CHUNK)],
                                    sems_o[slot])
    for h in out_h.values():
        h.wait()


@jax.jit
def _run(token_ids, token_table, pos_table):
    mesh = plsc.VectorSubcoreMesh(core_axis_name="c", subcore_axis_name="s")
    run = functools.partial(
        pl.kernel,
        out_type=jax.ShapeDtypeStruct((BATCH, SEQ, D), jnp.float32),
        mesh=mesh,
        scratch_types=[
            pltpu.VMEM((SEQ, D), jnp.float32),
            pltpu.VMEM((NBUF, CHUNK, SEQ), jnp.int32),
            pltpu.VMEM((NBUF, CHUNK, SEQ, D), jnp.float32),
            pltpu.SemaphoreType.DMA,
            pltpu.SemaphoreType.DMA,
            pltpu.SemaphoreType.DMA,
            pltpu.SemaphoreType.DMA,
        ],
        compiler_params=pltpu.CompilerParams(use_tc_tiling_on_sc=False),
    )(_embed_kernel)
    return run(token_ids.astype(jnp.int32), token_table, pos_table)


_printed = [False]


def kernel(token_ids, token_table, pos_table):
    if not _printed[0]:
        _printed[0] = True
        for name, a in (("ids", token_ids), ("table", token_table),
                        ("pos", pos_table)):
            try:
                print(f"[layout-probe] {name}: {a.format}", flush=True)
            except Exception as e:
                print(f"[layout-probe] {name}: err {e}", flush=True)
    return _run(token_ids, token_table, pos_table)


# speedup vs baseline: 1.2087x; 1.1238x over previous
import functools

import jax
import jax.numpy as jnp
from jax import lax
from jax.experimental import pallas as pl
from jax.experimental.pallas import tpu as pltpu
from jax.experimental.pallas import tpu_sc as plsc

NUM_CORES = 2
NUM_SUBCORES = 16
NUM_WORKERS = NUM_CORES * NUM_SUBCORES
LANES = 16

BATCH = 1024
SEQ = 200
D = 64
DPAD = 128
SPLITS = ((0, 104), (104, 96))
B_PER_W = BATCH // NUM_WORKERS
CHUNK = 2
NBUF = 2
NSTEPS = B_PER_W // CHUNK


def _embed_kernel(ids_hbm, table_hbm, pos_hbm, out_hbm,
                  pos_v, idx_v, tok_v, sem_g0, sem_g1, sem_o0, sem_o1):
    sems_g = (sem_g0, sem_g1)
    sems_o = (sem_o0, sem_o1)
    wid = lax.axis_index("s") * NUM_CORES + lax.axis_index("c")
    base = wid * B_PER_W

    pltpu.sync_copy(pos_hbm.at[pl.ds(0, SEQ)], pos_v)

    def stage(g):
        slot = g % NBUF
        b0 = base + g * CHUNK
        pltpu.sync_copy(ids_hbm.at[pl.ds(b0, CHUNK)], idx_v.at[slot])
        handles = []
        for c in range(CHUNK):
            for off, n in SPLITS:
                handles.append(pltpu.async_copy(
                    table_hbm.at[idx_v.at[slot, c, pl.ds(off, n)]],
                    tok_v.at[slot, c, pl.ds(off, n)],
                    sems_g[slot]))
        return handles

    gather_h = {0: stage(0)}
    out_h = {}
    for g in range(NSTEPS):
        slot = g % NBUF
        if g + 1 < NSTEPS:
            if g + 1 - NBUF >= 0:
                out_h.pop(g + 1 - NBUF).wait()
            gather_h[g + 1] = stage(g + 1)
        for h in gather_h.pop(g):
            h.wait()

        tok = tok_v.at[slot]

        def add_row(i, carry):
            for j in range(D // LANES):
                sl = pl.ds(j * LANES, LANES)
                p = pos_v[i, sl]
                for c in range(CHUNK):
                    plsc.addupdate(tok.at[c, i, sl], p)
            return carry

        lax.fori_loop(0, SEQ, add_row, 0)

        b0 = base + g * CHUNK
        out_h[g] = pltpu.async_copy(tok, out_hbm.at[pl.ds(b0, CHUNK)],
                                    sems_o[slot])
    for h in out_h.values():
        h.wait()


@jax.jit
def kernel(token_ids, token_table, pos_table):
    table128 = jnp.pad(token_table, ((0, 0), (0, DPAD - D)))
    mesh = plsc.VectorSubcoreMesh(core_axis_name="c", subcore_axis_name="s")
    run = functools.partial(
        pl.kernel,
        out_type=jax.ShapeDtypeStruct((BATCH, SEQ, DPAD), jnp.float32),
        mesh=mesh,
        scratch_types=[
            pltpu.VMEM((SEQ, D), jnp.float32),
            pltpu.VMEM((NBUF, CHUNK, SEQ), jnp.int32),
            pltpu.VMEM((NBUF, CHUNK, SEQ, DPAD), jnp.float32),
            pltpu.SemaphoreType.DMA,
            pltpu.SemaphoreType.DMA,
            pltpu.SemaphoreType.DMA,
            pltpu.SemaphoreType.DMA,
        ],
        compiler_params=pltpu.CompilerParams(use_tc_tiling_on_sc=False),
    )(_embed_kernel)
    out = run(token_ids.astype(jnp.int32), table128, pos_table)
    return out[:, :, :D]
